# baseline (speedup 1.0000x reference)
import jax
import jax.numpy as jnp
import numpy as np
from jax.experimental import pallas as pl
from jax.experimental.pallas import tpu as pltpu

_B = 128
_D = 2048
_A = 1000
_KBLK = 512
_NK = 2 * _D // _KBLK


def _threefry2x32_np(k0, k1, x0, x1):
    ks0 = np.uint32(k0)
    ks1 = np.uint32(k1)
    ks2 = np.uint32(ks0 ^ ks1 ^ np.uint32(0x1BD11BDA))
    ks = [ks0, ks1, ks2]
    rot = [[13, 15, 26, 6], [17, 29, 16, 24]]
    x0 = x0 + ks0
    x1 = x1 + ks1
    for r in range(5):
        for ri in rot[r % 2]:
            x0 = x0 + x1
            x1 = (x1 << np.uint32(ri)) | (x1 >> np.uint32(32 - ri))
            x1 = x1 ^ x0
        x0 = x0 + ks[(r + 1) % 3]
        x1 = x1 + ks[(r + 2) % 3] + np.uint32(r + 1)
    return x0, x1


def _gumbel_const(shape, seed):
    n = int(np.prod(shape))
    idx = np.arange(n, dtype=np.uint64)
    c_hi = (idx >> np.uint64(32)).astype(np.uint32)
    c_lo = (idx & np.uint64(0xFFFFFFFF)).astype(np.uint32)
    k0 = np.uint32(seed >> 32)
    k1 = np.uint32(seed & 0xFFFFFFFF)
    with np.errstate(over="ignore"):
        r0, r1 = _threefry2x32_np(k0, k1, c_hi, c_lo)
    bits = r0 ^ r1
    fb = (bits >> np.uint32(9)) | np.uint32(0x3F800000)
    u = fb.view(np.float32) - np.float32(1.0)
    tiny = np.float32(np.finfo(np.float32).tiny)
    u = u * (np.float32(1.0) - tiny) + tiny
    u = np.maximum(tiny, u)
    return (-np.log(-np.log(u))).astype(np.float32).reshape(shape)


_G = _gumbel_const((_B, _A), 42)


def _body(x_ref, w_ref, b_ref, g_ref,
          act_ref, nlp_ref, ent_ref, y_s):
    k = pl.program_id(0)

    part = jnp.dot(x_ref[...], w_ref[...], preferred_element_type=jnp.float32)

    @pl.when(k == 0)
    def _init():
        y_s[...] = part

    @pl.when(k > 0)
    def _acc():
        y_s[...] = y_s[...] + part

    @pl.when(k == _NK - 1)
    def _final():
        y = y_s[...] + b_ref[...]
        m = jnp.max(y, axis=1, keepdims=True)
        e = jnp.exp(y - m)
        s = jnp.sum(e, axis=1, keepdims=True)
        t = jnp.sum(y * e, axis=1, keepdims=True)
        logz = m + jnp.log(s)

        z = y + g_ref[...]
        bv = jnp.max(z, axis=1, keepdims=True)
        cols = jax.lax.broadcasted_iota(jnp.int32, (_B, _A), 1)
        bi = jnp.min(jnp.where(z == bv, cols, jnp.int32(2**30)),
                     axis=1, keepdims=True)
        ya = jnp.sum(jnp.where(cols == bi, y, 0.0), axis=1, keepdims=True)

        act_ref[...] = bi
        nlp_ref[...] = logz - ya
        ent_ref[...] = logz - t / s


def kernel(x, W, b):
    g = jnp.asarray(_G)
    b2 = b.reshape(1, _A)
    nxb = _D // _KBLK
    act, nlp, ent = pl.pallas_call(
        _body,
        grid=(_NK,),
        in_specs=[
            pl.BlockSpec((_B, _KBLK), lambda k: (0, k % nxb)),
            pl.BlockSpec((_KBLK, _A), lambda k: (k, 0)),
            pl.BlockSpec((1, _A), lambda k: (0, 0)),
            pl.BlockSpec((_B, _A), lambda k: (0, 0)),
        ],
        out_specs=[
            pl.BlockSpec((_B, 1), lambda k: (0, 0)),
            pl.BlockSpec((_B, 1), lambda k: (0, 0)),
            pl.BlockSpec((_B, 1), lambda k: (0, 0)),
        ],
        out_shape=[
            jax.ShapeDtypeStruct((_B, 1), jnp.int32),
            jax.ShapeDtypeStruct((_B, 1), jnp.float32),
            jax.ShapeDtypeStruct((_B, 1), jnp.float32),
        ],
        scratch_shapes=[
            pltpu.VMEM((_B, _A), jnp.float32),
        ],
    )(x, W, b2, g)
    return (act.reshape(_B), nlp.reshape(_B), ent.reshape(_B))


# speedup vs baseline: 1.0603x; 1.0603x over previous
import jax
import jax.numpy as jnp
import numpy as np
from jax.experimental import pallas as pl
from jax.experimental.pallas import tpu as pltpu

_B = 128
_D = 2048
_A = 1000
_KBLK = 256
_NWAY = 4
_KSTEP = _KBLK * _NWAY
_NK = 2 * _D // _KSTEP


def _threefry2x32_np(k0, k1, x0, x1):
    ks0 = np.uint32(k0)
    ks1 = np.uint32(k1)
    ks2 = np.uint32(ks0 ^ ks1 ^ np.uint32(0x1BD11BDA))
    ks = [ks0, ks1, ks2]
    rot = [[13, 15, 26, 6], [17, 29, 16, 24]]
    x0 = x0 + ks0
    x1 = x1 + ks1
    for r in range(5):
        for ri in rot[r % 2]:
            x0 = x0 + x1
            x1 = (x1 << np.uint32(ri)) | (x1 >> np.uint32(32 - ri))
            x1 = x1 ^ x0
        x0 = x0 + ks[(r + 1) % 3]
        x1 = x1 + ks[(r + 2) % 3] + np.uint32(r + 1)
    return x0, x1


def _gumbel_const(shape, seed):
    n = int(np.prod(shape))
    idx = np.arange(n, dtype=np.uint64)
    c_hi = (idx >> np.uint64(32)).astype(np.uint32)
    c_lo = (idx & np.uint64(0xFFFFFFFF)).astype(np.uint32)
    k0 = np.uint32(seed >> 32)
    k1 = np.uint32(seed & 0xFFFFFFFF)
    with np.errstate(over="ignore"):
        r0, r1 = _threefry2x32_np(k0, k1, c_hi, c_lo)
    bits = r0 ^ r1
    fb = (bits >> np.uint32(9)) | np.uint32(0x3F800000)
    u = fb.view(np.float32) - np.float32(1.0)
    tiny = np.float32(np.finfo(np.float32).tiny)
    u = u * (np.float32(1.0) - tiny) + tiny
    u = np.maximum(tiny, u)
    return (-np.log(-np.log(u))).astype(np.float32).reshape(shape)


_G = _gumbel_const((_B, _A), 42)


def _body(x_ref, w0_ref, w1_ref, w2_ref, w3_ref, b_ref, g_ref,
          act_ref, nlp_ref, ent_ref, y_s):
    k = pl.program_id(0)

    part = jnp.dot(x_ref[:, 0 * _KBLK:1 * _KBLK], w0_ref[...],
                   preferred_element_type=jnp.float32)
    part += jnp.dot(x_ref[:, 1 * _KBLK:2 * _KBLK], w1_ref[...],
                    preferred_element_type=jnp.float32)
    part += jnp.dot(x_ref[:, 2 * _KBLK:3 * _KBLK], w2_ref[...],
                    preferred_element_type=jnp.float32)
    part += jnp.dot(x_ref[:, 3 * _KBLK:4 * _KBLK], w3_ref[...],
                    preferred_element_type=jnp.float32)

    @pl.when(k == 0)
    def _init():
        y_s[...] = part

    @pl.when(k > 0)
    def _acc():
        y_s[...] = y_s[...] + part

    @pl.when(k == _NK - 1)
    def _final():
        y = y_s[...] + b_ref[...]
        m = jnp.max(y, axis=1, keepdims=True)
        e = jnp.exp(y - m)
        s = jnp.sum(e, axis=1, keepdims=True)
        t = jnp.sum(y * e, axis=1, keepdims=True)
        logz = m + jnp.log(s)

        z = y + g_ref[...]
        bv = jnp.max(z, axis=1, keepdims=True)
        cols = jax.lax.broadcasted_iota(jnp.int32, (_B, _A), 1)
        bi = jnp.min(jnp.where(z == bv, cols, jnp.int32(2**30)),
                     axis=1, keepdims=True)
        ya = jnp.sum(jnp.where(cols == bi, y, 0.0), axis=1, keepdims=True)

        act_ref[...] = bi
        nlp_ref[...] = logz - ya
        ent_ref[...] = logz - t / s


def kernel(x, W, b):
    g = jnp.asarray(_G)
    b2 = b.reshape(1, _A)
    nxb = _D // _KSTEP
    w_specs = [
        pl.BlockSpec((_KBLK, _A), lambda k, i=i: (k * _NWAY + i, 0))
        for i in range(_NWAY)
    ]
    act, nlp, ent = pl.pallas_call(
        _body,
        grid=(_NK,),
        in_specs=[
            pl.BlockSpec((_B, _KSTEP), lambda k: (0, k % nxb)),
            *w_specs,
            pl.BlockSpec((1, _A), lambda k: (0, 0)),
            pl.BlockSpec((_B, _A), lambda k: (0, 0)),
        ],
        out_specs=[
            pl.BlockSpec((_B, 1), lambda k: (0, 0)),
            pl.BlockSpec((_B, 1), lambda k: (0, 0)),
            pl.BlockSpec((_B, 1), lambda k: (0, 0)),
        ],
        out_shape=[
            jax.ShapeDtypeStruct((_B, 1), jnp.int32),
            jax.ShapeDtypeStruct((_B, 1), jnp.float32),
            jax.ShapeDtypeStruct((_B, 1), jnp.float32),
        ],
        scratch_shapes=[
            pltpu.VMEM((_B, _A), jnp.float32),
        ],
    )(x, W, W, W, W, b2, g)
    return (act.reshape(_B), nlp.reshape(_B), ent.reshape(_B))
